# baseline (speedup 1.0000x reference)
import functools

import jax
import jax.numpy as jnp
from jax import lax
from jax.experimental import pallas as pl
from jax.experimental.pallas import tpu as pltpu
from jax.experimental.pallas import tpu_sc as plsc

_NSET = 3
_CROWS = 8


def _make_sc_kernel(B, T, C, NC, NS):
    NW = NC * NS
    pos_rows_per_w = T // NW
    n_chunks = pos_rows_per_w // _CROWS

    mesh = plsc.VectorSubcoreMesh(core_axis_name="c", subcore_axis_name="s")

    scratch = []
    for _ in range(_NSET):
        scratch.append([pltpu.VMEM((_CROWS, C), jnp.float32) for _ in range(B)])
        scratch.append(pltpu.VMEM((_CROWS, C), jnp.float32))
        scratch.append(pltpu.SemaphoreType.DMA)
        scratch.append(pltpu.SemaphoreType.DMA)
        scratch.append(pltpu.SemaphoreType.DMA)

    @functools.partial(
        pl.kernel,
        out_type=jax.ShapeDtypeStruct((B, T, C), jnp.float32),
        mesh=mesh,
        scratch_types=scratch,
    )
    def body(x_3d, pos_hbm, out_3d, *sets):
        x_hbm = x_3d.reshape(B * T, C)
        out_hbm = out_3d.reshape(B * T, C)
        w = lax.axis_index("s") * NC + lax.axis_index("c")
        pr0 = w * pos_rows_per_w

        ld_descs = {}
        st_descs = {}

        def issue_loads(c):
            xbufs, pbuf, xsem, psem, _ = sets[5 * (c % _NSET):5 * (c % _NSET) + 5]
            row = pr0 + c * _CROWS
            descs = [
                pltpu.async_copy(
                    x_hbm.at[pl.ds(b * T + row, _CROWS)], xbufs[b], xsem
                )
                for b in range(B)
            ]
            descs.append(
                pltpu.async_copy(pos_hbm.at[pl.ds(row, _CROWS)], pbuf, psem)
            )
            ld_descs[c] = descs

        for c in range(min(_NSET - 1, n_chunks)):
            issue_loads(c)

        for c in range(n_chunks):
            s = c % _NSET
            xbufs, pbuf, _, _, stsem = sets[5 * s:5 * s + 5]
            for d in ld_descs.pop(c):
                d.wait()

            a0, a1, a2, a3 = xbufs

            @plsc.parallel_loop(0, _CROWS * C, step=16, unroll=4)
            def _(i):
                r = lax.shift_right_logical(i, 10)
                ds = pl.ds(pl.multiple_of(lax.bitwise_and(i, C - 1), 16), 16)
                pv = pbuf[r, ds]
                plsc.addupdate(a0.at[r, ds], pv)
                plsc.addupdate(a1.at[r, ds], pv)
                plsc.addupdate(a2.at[r, ds], pv)
                plsc.addupdate(a3.at[r, ds], pv)

            row = pr0 + c * _CROWS
            st_descs[c] = [
                pltpu.async_copy(
                    xbufs[b], out_hbm.at[pl.ds(b * T + row, _CROWS)], stsem
                )
                for b in range(B)
            ]

            nxt = c + _NSET - 1
            if nxt < n_chunks:
                for d in st_descs.pop(c - 1, ()):
                    d.wait()
                issue_loads(nxt)

        for c in sorted(st_descs):
            for d in st_descs[c]:
                d.wait()

    return body


def kernel(x, pos_emb):
    B, T, C = x.shape
    info = plsc.get_sparse_core_info()
    fn = _make_sc_kernel(B, T, C, info.num_cores, info.num_subcores)
    return fn(x, pos_emb)


# speedup vs baseline: 1.0223x; 1.0223x over previous
import functools

import jax
import jax.numpy as jnp
from jax import lax
from jax.experimental import pallas as pl
from jax.experimental.pallas import tpu as pltpu
from jax.experimental.pallas import tpu_sc as plsc

_NSET = 3
_CROWS = 8


def _make_sc_kernel(B, T, C, NC, NS):
    NW = NC * NS
    pos_rows_per_w = T // NW
    n_chunks = pos_rows_per_w // _CROWS

    mesh = plsc.VectorSubcoreMesh(core_axis_name="c", subcore_axis_name="s")

    scratch = []
    for _ in range(_NSET):
        scratch.append(pltpu.VMEM((B, _CROWS, C), jnp.float32))
        scratch.append(pltpu.VMEM((_CROWS, C), jnp.float32))
        scratch.append(pltpu.SemaphoreType.DMA)
        scratch.append(pltpu.SemaphoreType.DMA)
        scratch.append(pltpu.SemaphoreType.DMA)

    @functools.partial(
        pl.kernel,
        out_type=jax.ShapeDtypeStruct((B, T, C), jnp.float32),
        mesh=mesh,
        scratch_types=scratch,
    )
    def body(x_hbm, pos_hbm, out_hbm, *sets):
        w = lax.axis_index("s") * NC + lax.axis_index("c")
        pr0 = w * pos_rows_per_w

        ld_descs = {}
        st_descs = {}

        def issue_loads(c):
            xbuf, pbuf, xsem, psem, _ = sets[5 * (c % _NSET):5 * (c % _NSET) + 5]
            rows = pl.ds(pr0 + c * _CROWS, _CROWS)
            ld_descs[c] = [
                pltpu.async_copy(x_hbm.at[:, rows, :], xbuf, xsem),
                pltpu.async_copy(pos_hbm.at[rows], pbuf, psem),
            ]

        for c in range(min(_NSET - 1, n_chunks)):
            issue_loads(c)

        for c in range(n_chunks):
            s = c % _NSET
            xbuf, pbuf, _, _, stsem = sets[5 * s:5 * s + 5]
            for d in ld_descs.pop(c):
                d.wait()

            @plsc.parallel_loop(0, _CROWS * C, step=16, unroll=4)
            def _(i):
                r = lax.shift_right_logical(i, 10)
                ds = pl.ds(pl.multiple_of(lax.bitwise_and(i, C - 1), 16), 16)
                pv = pbuf[r, ds]
                for b in range(B):
                    plsc.addupdate(xbuf.at[b, r, ds], pv)

            rows = pl.ds(pr0 + c * _CROWS, _CROWS)
            st_descs[c] = pltpu.async_copy(xbuf, out_hbm.at[:, rows, :], stsem)

            nxt = c + _NSET - 1
            if nxt < n_chunks:
                if c - 1 in st_descs:
                    st_descs.pop(c - 1).wait()
                issue_loads(nxt)

        for c in sorted(st_descs):
            st_descs[c].wait()

    return body


def kernel(x, pos_emb):
    B, T, C = x.shape
    info = plsc.get_sparse_core_info()
    fn = _make_sc_kernel(B, T, C, info.num_cores, info.num_subcores)
    return fn(x, pos_emb)
